# baseline (speedup 1.0000x reference)
import functools

import jax
import jax.numpy as jnp
from jax import lax
from jax.experimental import pallas as pl
from jax.experimental.pallas import tpu as pltpu
from jax.experimental.pallas import tpu_sc as plsc

N = 10000
E = 320000
D = 128
TED = 32
R = 8
NBASIS = 8
OUT = 128
NT = 8
IN = D + TED

NC = 2
NS = 16
L = 16
NW = NC * NS

EROWS = E // 128
CROWS2 = 2560
CH = 80
NCHUNK = E // (NW * CH)
EPT = E // NW
CROWS = 16
NCCH = CROWS2 // NW // CROWS
BN = 200
NBLK = N // BN
HSLOT = N * R
HSIZE = 80128
HPT = HSIZE // NS

_f32 = jnp.float32
_i32 = jnp.int32


def _eidx_body(src_ref, dst_ref, et_ref, gidx_ref, cidx_ref, cidxb_ref):
    et = et_ref[...]
    gidx_ref[...] = et * N + src_ref[...]
    cidx = dst_ref[...] * R + et
    cidx_ref[...] = cidx
    cidxb_ref[pl.ds(0, EROWS), :] = cidx
    cidxb_ref[pl.ds(EROWS, CROWS2 - EROWS), :] = jnp.full(
        (CROWS2 - EROWS, 128), HSLOT, _i32)


def _eidx_call(src2, dst2, et2):
    return pl.pallas_call(
        _eidx_body,
        grid=(1,),
        in_specs=[pl.BlockSpec((EROWS, 128), lambda i: (0, 0))] * 3,
        out_specs=[
            pl.BlockSpec((EROWS, 128), lambda i: (0, 0)),
            pl.BlockSpec((EROWS, 128), lambda i: (0, 0)),
            pl.BlockSpec((CROWS2, 128), lambda i: (0, 0)),
        ],
        out_shape=[
            jax.ShapeDtypeStruct((EROWS, 128), _i32),
            jax.ShapeDtypeStruct((EROWS, 128), _i32),
            jax.ShapeDtypeStruct((CROWS2, 128), _i32),
        ],
    )(src2, dst2, et2)


def _dense_body(ids_ref, x_ref, temb_ref, comp_ref, basis_ref, root_ref,
                bias_ref, hx_ref, self_ref, w_s):
    i = pl.program_id(0)

    @pl.when(i == 0)
    def _():
        for r in range(R):
            acc = comp_ref[r, 0] * basis_ref[0]
            for b in range(1, NBASIS):
                acc = acc + comp_ref[r, b] * basis_ref[b]
            w_s[r] = acc

    ids = ids_ref[0]
    iota_t = lax.broadcasted_iota(_i32, (1, NT), 1)
    onehot = (ids == iota_t).astype(_f32)
    temb = jnp.dot(onehot, temb_ref[...], preferred_element_type=_f32)
    xb = x_ref[...]
    for r in range(R):
        hx_ref[r] = (jnp.dot(xb, w_s[r, :D, :], preferred_element_type=_f32)
                     + jnp.dot(temb, w_s[r, D:, :], preferred_element_type=_f32))
    self_ref[...] = (jnp.dot(xb, root_ref[:D, :], preferred_element_type=_f32)
                     + jnp.dot(temb, root_ref[D:, :], preferred_element_type=_f32)
                     + bias_ref[...])


def _dense_call(ids3, x, type_emb, comp, basis, root, bias2):
    return pl.pallas_call(
        _dense_body,
        grid=(NBLK,),
        in_specs=[
            pl.BlockSpec((1, BN, 1), lambda i: (i, 0, 0)),
            pl.BlockSpec((BN, D), lambda i: (i, 0)),
            pl.BlockSpec((NT, TED), lambda i: (0, 0)),
            pl.BlockSpec(memory_space=pltpu.SMEM),
            pl.BlockSpec((NBASIS, IN, OUT), lambda i: (0, 0, 0)),
            pl.BlockSpec((IN, OUT), lambda i: (0, 0)),
            pl.BlockSpec((1, OUT), lambda i: (0, 0)),
        ],
        out_specs=[
            pl.BlockSpec((R, BN, OUT), lambda i: (0, i, 0)),
            pl.BlockSpec((BN, OUT), lambda i: (i, 0)),
        ],
        out_shape=[
            jax.ShapeDtypeStruct((R, N, OUT), _f32),
            jax.ShapeDtypeStruct((N, OUT), _f32),
        ],
        scratch_shapes=[pltpu.VMEM((R, IN, OUT), _f32)],
    )(ids3, x, type_emb, comp, basis, root, bias2)


def _sc_count_body(cidxb_hbm, cnt_hbm, cnt_sp, cb0, cb1, ones_v, zbuf, ca_v,
                   si0, si1, ss0, ss1):
    c = lax.axis_index("c")
    s = lax.axis_index("s")
    wid = c * NS + s

    cb = (cb0, cb1)
    sem_i = (si0, si1)
    sem_s = (ss0, ss1)

    zeros16 = jnp.zeros((L,), _f32)
    ones16 = jnp.ones((L,), _f32)

    def _zb(k, _):
        zbuf[pl.ds(k * L, L)] = zeros16
        return _
    lax.fori_loop(0, 2048 // L, _zb, None)

    for k in range(128 // L):
        ones_v[pl.ds(k * L, L)] = ones16

    cbase = s * HPT
    pltpu.sync_copy(zbuf.at[pl.ds(0, 2000)], cnt_sp.at[pl.ds(cbase, 2000)])
    pltpu.sync_copy(zbuf.at[pl.ds(0, 2000)], cnt_sp.at[pl.ds(cbase + 2000, 2000)])
    pltpu.sync_copy(zbuf.at[pl.ds(0, 1008)], cnt_sp.at[pl.ds(cbase + 4000, 1008)])

    plsc.subcore_barrier()

    crow0 = wid * (CROWS * NCCH)

    def _fire_cnt(j, p):
        pltpu.async_copy(cidxb_hbm.at[pl.ds(crow0 + j * CROWS, CROWS)],
                         cb[p], sem_i[p])

    def _do_cnt(p):
        pltpu.make_async_copy(cidxb_hbm.at[pl.ds(0, CROWS)], cb[p],
                              sem_i[p]).wait()
        for jj in range(CROWS):
            pltpu.async_copy(ones_v, cnt_sp.at[cb[p].at[jj]], sem_s[p],
                             add=True)
        for jj in range(CROWS):
            pltpu.make_async_copy(ones_v, cnt_sp.at[cb[p].at[jj]],
                                  sem_s[p]).wait()

    _fire_cnt(0, 0)
    _fire_cnt(1, 1)
    _do_cnt(0)
    _fire_cnt(2, 0)
    _do_cnt(1)
    _fire_cnt(3, 1)
    _do_cnt(0)
    _fire_cnt(4, 0)
    _do_cnt(1)
    _do_cnt(0)

    plsc.subcore_barrier()

    pltpu.sync_copy(cnt_sp.at[pl.ds(cbase, HPT)], ca_v)
    pltpu.sync_copy(ca_v, cnt_hbm.at[pl.ds(c * HSIZE + cbase, HPT)])


_sc_count = functools.partial(
    pl.kernel,
    out_type=jax.ShapeDtypeStruct((NC * HSIZE,), _f32),
    mesh=plsc.VectorSubcoreMesh(core_axis_name="c", subcore_axis_name="s"),
    compiler_params=pltpu.CompilerParams(needs_layout_passes=False),
    scratch_types=[
        pltpu.VMEM_SHARED((HSIZE,), _f32),
        pltpu.VMEM((CROWS, 128), _i32),
        pltpu.VMEM((CROWS, 128), _i32),
        pltpu.VMEM((128,), _f32),
        pltpu.VMEM((2048,), _f32),
        pltpu.VMEM((HPT,), _f32),
        pltpu.SemaphoreType.DMA,
        pltpu.SemaphoreType.DMA,
        pltpu.SemaphoreType.DMA,
        pltpu.SemaphoreType.DMA,
    ],
)(_sc_count_body)


def _sc_edges_body(gidx_hbm, cidx_hbm, cnt_hbm, hx_hbm, out_hbm,
                   cnt_sp, acc_sp,
                   rows0, rows1, rows2, gq0, gq1, gq2, cq0, cq1, cq2,
                   scl0, scl1, scl2, dst0, dst1, dst2,
                   ca0, ca1,
                   si0, si1, si2, sr0, sr1, sr2, sc0, sc1, sc2, ss0, ss1, ss2):
    c = lax.axis_index("c")
    s = lax.axis_index("s")
    wid = c * NS + s
    ebase = wid * EPT

    rows = (rows0, rows1, rows2)
    gq = (gq0, gq1, gq2)
    cq = (cq0, cq1, cq2)
    scl = (scl0, scl1, scl2)
    dstb = (dst0, dst1, dst2)
    sem_i = (si0, si1, si2)
    sem_r = (sr0, sr1, sr2)
    sem_c = (sc0, sc1, sc2)
    sem_s = (ss0, ss1, ss2)

    zeros16 = jnp.zeros((L,), _f32)

    cbase = s * HPT
    pltpu.async_copy(cnt_hbm.at[pl.ds(cbase, HPT)], ca0, si0)
    pltpu.async_copy(cnt_hbm.at[pl.ds(HSIZE + cbase, HPT)], ca1, si1)

    def _zr(e, _):
        for k in range(OUT // L):
            rows0[e, pl.ds(k * L, L)] = zeros16
        return _
    lax.fori_loop(0, CH, _zr, None)

    pltpu.make_async_copy(cnt_hbm.at[pl.ds(0, HPT)], ca0, si0).wait()
    pltpu.make_async_copy(cnt_hbm.at[pl.ds(0, HPT)], ca1, si1).wait()

    def _csum(k, _):
        sl = pl.ds(k * L, L)
        ca0[sl] = ca0[sl] + ca1[sl]
        return _
    lax.fori_loop(0, HPT // L, _csum, None)
    pltpu.sync_copy(ca0, cnt_sp.at[pl.ds(cbase, HPT)])

    @pl.when(s < 10)
    def _():
        rbase = s * 1000
        for j in range(12):
            pltpu.sync_copy(rows0.at[pl.ds(0, CH)],
                            acc_sp.at[pl.ds(rbase + j * CH, CH)])
        pltpu.sync_copy(rows0.at[pl.ds(0, 40)],
                        acc_sp.at[pl.ds(rbase + 960, 40)])

    plsc.subcore_barrier()

    def _fire_idx(t, p):
        base = ebase + t * CH
        pltpu.async_copy(gidx_hbm.at[pl.ds(base, CH)], gq[p], sem_i[p])
        pltpu.async_copy(cidx_hbm.at[pl.ds(base, CH)], cq[p], sem_i[p])

    def _wait_idx(p):
        pltpu.make_async_copy(gidx_hbm.at[pl.ds(0, CH)], gq[p], sem_i[p]).wait()
        pltpu.make_async_copy(cidx_hbm.at[pl.ds(0, CH)], cq[p], sem_i[p]).wait()

    def _fire_gather(p):
        pltpu.async_copy(hx_hbm.at[gq[p]], rows[p], sem_r[p])
        pltpu.async_copy(cnt_sp.at[cq[p]], scl[p], sem_c[p])

    def _wait_gather(p):
        pltpu.make_async_copy(hx_hbm.at[gq[p]], rows[p], sem_r[p]).wait()
        pltpu.make_async_copy(cnt_sp.at[cq[p]], scl[p], sem_c[p]).wait()

    def _fire_scatter(p):
        pltpu.async_copy(rows[p], acc_sp.at[dstb[p]], sem_s[p], add=True)

    def _wait_scatter(p):
        pltpu.make_async_copy(rows[p], acc_sp.at[dstb[p]], sem_s[p]).wait()

    def _compute(p):
        for k in range(CH // L):
            sl = pl.ds(k * L, L)
            scl[p][sl] = 1.0 / scl[p][sl]
            dstb[p][sl] = lax.shift_right_logical(cq[p][sl], 3)

    def _scale_rows(p):
        @plsc.parallel_loop(0, CH, 1, unroll=8)
        def _one(e):
            svec = plsc.load_gather(scl[p], [jnp.full((L,), e, _i32)])
            for k in range(OUT // L):
                sl2 = pl.ds(k * L, L)
                rows[p][e, sl2] = rows[p][e, sl2] * svec

    def _slot(t, b0, b1, b2, w_i=True, w_sc=True, f_g=True, f_i=True):
        if w_i:
            _wait_idx(b1)
        if w_sc:
            _wait_scatter(b1)
        if f_g:
            _fire_gather(b1)
        _wait_gather(b0)
        _compute(b0)
        if f_i:
            _fire_idx(t + 2, b2)
        _scale_rows(b0)
        _fire_scatter(b0)

    _fire_idx(0, 0)
    _fire_idx(1, 1)
    _wait_idx(0)
    _fire_gather(0)
    _slot(0, 0, 1, 2, w_sc=False)
    _slot(1, 1, 2, 0, w_sc=False)

    def _triple(i, _):
        t = 3 * i + 2
        _slot(t, 2, 0, 1)
        _slot(t + 1, 0, 1, 2)
        _slot(t + 2, 1, 2, 0)
        return _
    lax.fori_loop(0, 40, _triple, None)

    _slot(122, 2, 0, 1)
    _slot(123, 0, 1, 2, f_i=False)
    _slot(124, 1, 2, 0, w_i=False, f_g=False, f_i=False)
    _wait_scatter(0)
    _wait_scatter(1)

    plsc.subcore_barrier()

    @pl.when(s < 10)
    def _():
        rbase = s * 1000
        pltpu.sync_copy(acc_sp.at[pl.ds(rbase, 1000)],
                        out_hbm.at[pl.ds(c * N + rbase, 1000)])


_sc_edges = functools.partial(
    pl.kernel,
    out_type=jax.ShapeDtypeStruct((NC * N, OUT), _f32),
    mesh=plsc.VectorSubcoreMesh(core_axis_name="c", subcore_axis_name="s"),
    compiler_params=pltpu.CompilerParams(needs_layout_passes=False),
    scratch_types=[
        pltpu.VMEM_SHARED((HSIZE,), _f32),
        pltpu.VMEM_SHARED((N, OUT), _f32),
        pltpu.VMEM((CH, OUT), _f32),
        pltpu.VMEM((CH, OUT), _f32),
        pltpu.VMEM((CH, OUT), _f32),
        pltpu.VMEM((CH,), _i32),
        pltpu.VMEM((CH,), _i32),
        pltpu.VMEM((CH,), _i32),
        pltpu.VMEM((CH,), _i32),
        pltpu.VMEM((CH,), _i32),
        pltpu.VMEM((CH,), _i32),
        pltpu.VMEM((CH,), _f32),
        pltpu.VMEM((CH,), _f32),
        pltpu.VMEM((CH,), _f32),
        pltpu.VMEM((CH,), _i32),
        pltpu.VMEM((CH,), _i32),
        pltpu.VMEM((CH,), _i32),
        pltpu.VMEM((HPT,), _f32),
        pltpu.VMEM((HPT,), _f32),
        pltpu.SemaphoreType.DMA,
        pltpu.SemaphoreType.DMA,
        pltpu.SemaphoreType.DMA,
        pltpu.SemaphoreType.DMA,
        pltpu.SemaphoreType.DMA,
        pltpu.SemaphoreType.DMA,
        pltpu.SemaphoreType.DMA,
        pltpu.SemaphoreType.DMA,
        pltpu.SemaphoreType.DMA,
        pltpu.SemaphoreType.DMA,
        pltpu.SemaphoreType.DMA,
        pltpu.SemaphoreType.DMA,
    ],
)(_sc_edges_body)


def _finish_body(acc_ref, self_ref, out_ref):
    out_ref[...] = jnp.maximum(acc_ref[0] + acc_ref[1] + self_ref[...], 0.0)


def _finish_call(acc, selfterm):
    return pl.pallas_call(
        _finish_body,
        grid=(NBLK,),
        in_specs=[
            pl.BlockSpec((NC, BN, OUT), lambda i: (0, i, 0)),
            pl.BlockSpec((BN, OUT), lambda i: (i, 0)),
        ],
        out_specs=pl.BlockSpec((BN, OUT), lambda i: (i, 0)),
        out_shape=jax.ShapeDtypeStruct((N, OUT), _f32),
    )(acc, selfterm)


def kernel(x, node_type_ids, edge_index, edge_type, type_emb, basis, comp,
           root, bias):
    src2 = edge_index[0].astype(_i32).reshape(EROWS, 128)
    dst2 = edge_index[1].astype(_i32).reshape(EROWS, 128)
    et2 = edge_type.astype(_i32).reshape(EROWS, 128)
    gidx2, cidx2, cidxb = _eidx_call(src2, dst2, et2)
    cnt2 = _sc_count(cidxb)

    ids3 = node_type_ids.astype(_i32).reshape(NBLK, BN, 1)
    bias2 = bias.reshape(1, OUT)
    hx, selfterm = _dense_call(ids3, x, type_emb, comp, basis, root, bias2)

    acc = _sc_edges(gidx2.reshape(E), cidx2.reshape(E), cnt2,
                    hx.reshape(R * N, OUT))
    return _finish_call(acc.reshape(NC, N, OUT), selfterm)


# speedup vs baseline: 1.0037x; 1.0037x over previous
import functools

import jax
import jax.numpy as jnp
from jax import lax
from jax.experimental import pallas as pl
from jax.experimental.pallas import tpu as pltpu
from jax.experimental.pallas import tpu_sc as plsc

N = 10000
E = 320000
D = 128
TED = 32
R = 8
NBASIS = 8
OUT = 128
NT = 8
IN = D + TED

NC = 2
NS = 16
L = 16
NW = NC * NS

EROWS = E // 128
CROWS2 = 2560
CH = 80
NCHUNK = E // (NW * CH)
EPT = E // NW
CROWS = 16
NCCH = CROWS2 // NW // CROWS
BN = 200
NBLK = N // BN
HSLOT = N * R
HSIZE = 80128
HPT = HSIZE // NS

_f32 = jnp.float32
_i32 = jnp.int32


def _eidx_body(src_ref, dst_ref, et_ref, gidx_ref, cidx_ref, cidxb_ref):
    et = et_ref[...]
    gidx_ref[...] = et * N + src_ref[...]
    cidx = dst_ref[...] * R + et
    cidx_ref[...] = cidx
    cidxb_ref[pl.ds(0, EROWS), :] = cidx
    cidxb_ref[pl.ds(EROWS, CROWS2 - EROWS), :] = jnp.full(
        (CROWS2 - EROWS, 128), HSLOT, _i32)


def _eidx_call(src2, dst2, et2):
    return pl.pallas_call(
        _eidx_body,
        grid=(1,),
        in_specs=[pl.BlockSpec((EROWS, 128), lambda i: (0, 0))] * 3,
        out_specs=[
            pl.BlockSpec((EROWS, 128), lambda i: (0, 0)),
            pl.BlockSpec((EROWS, 128), lambda i: (0, 0)),
            pl.BlockSpec((CROWS2, 128), lambda i: (0, 0)),
        ],
        out_shape=[
            jax.ShapeDtypeStruct((EROWS, 128), _i32),
            jax.ShapeDtypeStruct((EROWS, 128), _i32),
            jax.ShapeDtypeStruct((CROWS2, 128), _i32),
        ],
    )(src2, dst2, et2)


def _dense_body(ids_ref, x_ref, temb_ref, comp_ref, basis_ref, root_ref,
                bias_ref, hx_ref, self_ref, w_s):
    i = pl.program_id(0)

    @pl.when(i == 0)
    def _():
        for r in range(R):
            acc = comp_ref[r, 0] * basis_ref[0]
            for b in range(1, NBASIS):
                acc = acc + comp_ref[r, b] * basis_ref[b]
            w_s[r] = acc

    ids = ids_ref[0]
    iota_t = lax.broadcasted_iota(_i32, (1, NT), 1)
    onehot = (ids == iota_t).astype(_f32)
    temb = jnp.dot(onehot, temb_ref[...], preferred_element_type=_f32)
    xb = x_ref[...]
    for r in range(R):
        hx_ref[r] = (jnp.dot(xb, w_s[r, :D, :], preferred_element_type=_f32)
                     + jnp.dot(temb, w_s[r, D:, :], preferred_element_type=_f32))
    self_ref[...] = (jnp.dot(xb, root_ref[:D, :], preferred_element_type=_f32)
                     + jnp.dot(temb, root_ref[D:, :], preferred_element_type=_f32)
                     + bias_ref[...])


def _dense_call(ids3, x, type_emb, comp, basis, root, bias2):
    return pl.pallas_call(
        _dense_body,
        grid=(NBLK,),
        in_specs=[
            pl.BlockSpec((1, BN, 1), lambda i: (i, 0, 0)),
            pl.BlockSpec((BN, D), lambda i: (i, 0)),
            pl.BlockSpec((NT, TED), lambda i: (0, 0)),
            pl.BlockSpec(memory_space=pltpu.SMEM),
            pl.BlockSpec((NBASIS, IN, OUT), lambda i: (0, 0, 0)),
            pl.BlockSpec((IN, OUT), lambda i: (0, 0)),
            pl.BlockSpec((1, OUT), lambda i: (0, 0)),
        ],
        out_specs=[
            pl.BlockSpec((R, BN, OUT), lambda i: (0, i, 0)),
            pl.BlockSpec((BN, OUT), lambda i: (i, 0)),
        ],
        out_shape=[
            jax.ShapeDtypeStruct((R, N, OUT), _f32),
            jax.ShapeDtypeStruct((N, OUT), _f32),
        ],
        scratch_shapes=[pltpu.VMEM((R, IN, OUT), _f32)],
    )(ids3, x, type_emb, comp, basis, root, bias2)


def _sc_count_body(cidxb_hbm, cnt_hbm, cnt_sp, cb0, cb1, ones_v, zbuf, ca_v,
                   si0, si1, ss0, ss1):
    c = lax.axis_index("c")
    s = lax.axis_index("s")
    wid = c * NS + s

    cb = (cb0, cb1)
    sem_i = (si0, si1)
    sem_s = (ss0, ss1)

    zeros16 = jnp.zeros((L,), _f32)
    ones16 = jnp.ones((L,), _f32)

    def _zb(k, _):
        zbuf[pl.ds(k * L, L)] = zeros16
        return _
    lax.fori_loop(0, 2048 // L, _zb, None)

    for k in range(128 // L):
        ones_v[pl.ds(k * L, L)] = ones16

    cbase = s * HPT
    pltpu.sync_copy(zbuf.at[pl.ds(0, 2000)], cnt_sp.at[pl.ds(cbase, 2000)])
    pltpu.sync_copy(zbuf.at[pl.ds(0, 2000)], cnt_sp.at[pl.ds(cbase + 2000, 2000)])
    pltpu.sync_copy(zbuf.at[pl.ds(0, 1008)], cnt_sp.at[pl.ds(cbase + 4000, 1008)])

    plsc.subcore_barrier()

    crow0 = wid * (CROWS * NCCH)

    def _fire_cnt(j, p):
        pltpu.async_copy(cidxb_hbm.at[pl.ds(crow0 + j * CROWS, CROWS)],
                         cb[p], sem_i[p])

    def _do_cnt(p):
        pltpu.make_async_copy(cidxb_hbm.at[pl.ds(0, CROWS)], cb[p],
                              sem_i[p]).wait()
        for jj in range(CROWS):
            pltpu.async_copy(ones_v, cnt_sp.at[cb[p].at[jj]], sem_s[p],
                             add=True)
        for jj in range(CROWS):
            pltpu.make_async_copy(ones_v, cnt_sp.at[cb[p].at[jj]],
                                  sem_s[p]).wait()

    _fire_cnt(0, 0)
    _fire_cnt(1, 1)
    _do_cnt(0)
    _fire_cnt(2, 0)
    _do_cnt(1)
    _fire_cnt(3, 1)
    _do_cnt(0)
    _fire_cnt(4, 0)
    _do_cnt(1)
    _do_cnt(0)

    plsc.subcore_barrier()

    pltpu.sync_copy(cnt_sp.at[pl.ds(cbase, HPT)], ca_v)
    pltpu.sync_copy(ca_v, cnt_hbm.at[pl.ds(c * HSIZE + cbase, HPT)])


_sc_count = functools.partial(
    pl.kernel,
    out_type=jax.ShapeDtypeStruct((NC * HSIZE,), _f32),
    mesh=plsc.VectorSubcoreMesh(core_axis_name="c", subcore_axis_name="s"),
    compiler_params=pltpu.CompilerParams(needs_layout_passes=False),
    scratch_types=[
        pltpu.VMEM_SHARED((HSIZE,), _f32),
        pltpu.VMEM((CROWS, 128), _i32),
        pltpu.VMEM((CROWS, 128), _i32),
        pltpu.VMEM((128,), _f32),
        pltpu.VMEM((2048,), _f32),
        pltpu.VMEM((HPT,), _f32),
        pltpu.SemaphoreType.DMA,
        pltpu.SemaphoreType.DMA,
        pltpu.SemaphoreType.DMA,
        pltpu.SemaphoreType.DMA,
    ],
)(_sc_count_body)


def _sc_edges_body(gidx_hbm, cidx_hbm, cnt_hbm, hx_hbm, out_hbm,
                   cnt_sp, acc_sp,
                   rows0, rows1, rows2, gq0, gq1, gq2, cq0, cq1, cq2,
                   scl0, scl1, scl2, dst0, dst1, dst2,
                   ca0, ca1,
                   si0, si1, si2, sr0, sr1, sr2, sc0, sc1, sc2, ss0, ss1, ss2):
    c = lax.axis_index("c")
    s = lax.axis_index("s")
    wid = c * NS + s
    ebase = wid * EPT

    rows = (rows0, rows1, rows2)
    gq = (gq0, gq1, gq2)
    cq = (cq0, cq1, cq2)
    scl = (scl0, scl1, scl2)
    dstb = (dst0, dst1, dst2)
    sem_i = (si0, si1, si2)
    sem_r = (sr0, sr1, sr2)
    sem_c = (sc0, sc1, sc2)
    sem_s = (ss0, ss1, ss2)

    zeros16 = jnp.zeros((L,), _f32)

    cbase = s * HPT
    pltpu.async_copy(cnt_hbm.at[pl.ds(cbase, HPT)], ca0, si0)
    pltpu.async_copy(cnt_hbm.at[pl.ds(HSIZE + cbase, HPT)], ca1, si1)

    def _zr(e, _):
        for k in range(OUT // L):
            rows0[e, pl.ds(k * L, L)] = zeros16
        return _
    lax.fori_loop(0, CH, _zr, None)

    pltpu.make_async_copy(cnt_hbm.at[pl.ds(0, HPT)], ca0, si0).wait()
    pltpu.make_async_copy(cnt_hbm.at[pl.ds(0, HPT)], ca1, si1).wait()

    def _csum(k, _):
        sl = pl.ds(k * L, L)
        ca0[sl] = ca0[sl] + ca1[sl]
        return _
    lax.fori_loop(0, HPT // L, _csum, None)
    pltpu.sync_copy(ca0, cnt_sp.at[pl.ds(cbase, HPT)])

    @pl.when(s < 10)
    def _():
        rbase = s * 1000
        for j in range(12):
            pltpu.sync_copy(rows0.at[pl.ds(0, CH)],
                            acc_sp.at[pl.ds(rbase + j * CH, CH)])
        pltpu.sync_copy(rows0.at[pl.ds(0, 40)],
                        acc_sp.at[pl.ds(rbase + 960, 40)])

    plsc.subcore_barrier()

    def _fire_idx(t, p):
        base = ebase + t * CH
        pltpu.async_copy(gidx_hbm.at[pl.ds(base, CH)], gq[p], sem_i[p])
        pltpu.async_copy(cidx_hbm.at[pl.ds(base, CH)], cq[p], sem_i[p])

    def _wait_idx(p):
        pltpu.make_async_copy(gidx_hbm.at[pl.ds(0, CH)], gq[p], sem_i[p]).wait()
        pltpu.make_async_copy(cidx_hbm.at[pl.ds(0, CH)], cq[p], sem_i[p]).wait()

    def _fire_gather(p):
        pltpu.async_copy(hx_hbm.at[gq[p]], rows[p], sem_r[p])
        pltpu.async_copy(cnt_sp.at[cq[p]], scl[p], sem_c[p])

    def _wait_gather(p):
        pltpu.make_async_copy(hx_hbm.at[gq[p]], rows[p], sem_r[p]).wait()
        pltpu.make_async_copy(cnt_sp.at[cq[p]], scl[p], sem_c[p]).wait()

    def _fire_scatter(p):
        pltpu.async_copy(rows[p], acc_sp.at[dstb[p]], sem_s[p], add=True)

    def _wait_scatter(p):
        pltpu.make_async_copy(rows[p], acc_sp.at[dstb[p]], sem_s[p]).wait()

    def _compute(p):
        for k in range(CH // L):
            sl = pl.ds(k * L, L)
            scl[p][sl] = 1.0 / scl[p][sl]
            dstb[p][sl] = lax.shift_right_logical(cq[p][sl], 3)

    def _scale_rows(p):
        @plsc.parallel_loop(0, CH, 1, unroll=4)
        def _one(e):
            svec = plsc.load_gather(scl[p], [jnp.full((L,), e, _i32)])
            for k in range(OUT // L):
                sl2 = pl.ds(k * L, L)
                rows[p][e, sl2] = rows[p][e, sl2] * svec

    def _slot(t, b0, b1, b2, w_i=True, w_sc=True, f_g=True, f_i=True):
        if w_i:
            _wait_idx(b1)
        if w_sc:
            _wait_scatter(b1)
        if f_g:
            _fire_gather(b1)
        _wait_gather(b0)
        _compute(b0)
        if f_i:
            _fire_idx(t + 2, b2)
        _scale_rows(b0)
        _fire_scatter(b0)

    _fire_idx(0, 0)
    _fire_idx(1, 1)
    _wait_idx(0)
    _fire_gather(0)
    _slot(0, 0, 1, 2, w_sc=False)
    _slot(1, 1, 2, 0, w_sc=False)

    def _triple(i, _):
        t = 3 * i + 2
        _slot(t, 2, 0, 1)
        _slot(t + 1, 0, 1, 2)
        _slot(t + 2, 1, 2, 0)
        return _
    lax.fori_loop(0, 40, _triple, None)

    _slot(122, 2, 0, 1)
    _slot(123, 0, 1, 2, f_i=False)
    _slot(124, 1, 2, 0, w_i=False, f_g=False, f_i=False)
    _wait_scatter(0)
    _wait_scatter(1)

    plsc.subcore_barrier()

    @pl.when(s < 10)
    def _():
        rbase = s * 1000
        pltpu.sync_copy(acc_sp.at[pl.ds(rbase, 1000)],
                        out_hbm.at[pl.ds(c * N + rbase, 1000)])


_sc_edges = functools.partial(
    pl.kernel,
    out_type=jax.ShapeDtypeStruct((NC * N, OUT), _f32),
    mesh=plsc.VectorSubcoreMesh(core_axis_name="c", subcore_axis_name="s"),
    compiler_params=pltpu.CompilerParams(needs_layout_passes=False),
    scratch_types=[
        pltpu.VMEM_SHARED((HSIZE,), _f32),
        pltpu.VMEM_SHARED((N, OUT), _f32),
        pltpu.VMEM((CH, OUT), _f32),
        pltpu.VMEM((CH, OUT), _f32),
        pltpu.VMEM((CH, OUT), _f32),
        pltpu.VMEM((CH,), _i32),
        pltpu.VMEM((CH,), _i32),
        pltpu.VMEM((CH,), _i32),
        pltpu.VMEM((CH,), _i32),
        pltpu.VMEM((CH,), _i32),
        pltpu.VMEM((CH,), _i32),
        pltpu.VMEM((CH,), _f32),
        pltpu.VMEM((CH,), _f32),
        pltpu.VMEM((CH,), _f32),
        pltpu.VMEM((CH,), _i32),
        pltpu.VMEM((CH,), _i32),
        pltpu.VMEM((CH,), _i32),
        pltpu.VMEM((HPT,), _f32),
        pltpu.VMEM((HPT,), _f32),
        pltpu.SemaphoreType.DMA,
        pltpu.SemaphoreType.DMA,
        pltpu.SemaphoreType.DMA,
        pltpu.SemaphoreType.DMA,
        pltpu.SemaphoreType.DMA,
        pltpu.SemaphoreType.DMA,
        pltpu.SemaphoreType.DMA,
        pltpu.SemaphoreType.DMA,
        pltpu.SemaphoreType.DMA,
        pltpu.SemaphoreType.DMA,
        pltpu.SemaphoreType.DMA,
        pltpu.SemaphoreType.DMA,
    ],
)(_sc_edges_body)


def _finish_body(acc_ref, self_ref, out_ref):
    out_ref[...] = jnp.maximum(acc_ref[0] + acc_ref[1] + self_ref[...], 0.0)


def _finish_call(acc, selfterm):
    return pl.pallas_call(
        _finish_body,
        grid=(NBLK,),
        in_specs=[
            pl.BlockSpec((NC, BN, OUT), lambda i: (0, i, 0)),
            pl.BlockSpec((BN, OUT), lambda i: (i, 0)),
        ],
        out_specs=pl.BlockSpec((BN, OUT), lambda i: (i, 0)),
        out_shape=jax.ShapeDtypeStruct((N, OUT), _f32),
    )(acc, selfterm)


def kernel(x, node_type_ids, edge_index, edge_type, type_emb, basis, comp,
           root, bias):
    src2 = edge_index[0].astype(_i32).reshape(EROWS, 128)
    dst2 = edge_index[1].astype(_i32).reshape(EROWS, 128)
    et2 = edge_type.astype(_i32).reshape(EROWS, 128)
    gidx2, cidx2, cidxb = _eidx_call(src2, dst2, et2)
    cnt2 = _sc_count(cidxb)

    ids3 = node_type_ids.astype(_i32).reshape(NBLK, BN, 1)
    bias2 = bias.reshape(1, OUT)
    hx, selfterm = _dense_call(ids3, x, type_emb, comp, basis, root, bias2)

    acc = _sc_edges(gidx2.reshape(E), cidx2.reshape(E), cnt2,
                    hx.reshape(R * N, OUT))
    return _finish_call(acc.reshape(NC, N, OUT), selfterm)
